# baseline (speedup 1.0000x reference)
import functools

import jax
import jax.numpy as jnp
from jax import lax
from jax.experimental import pallas as pl
from jax.experimental.pallas import tpu as pltpu
from jax.experimental.pallas import tpu_sc as plsc

_NC = 2
_NS = 16
_NW = _NC * _NS
_L = 16
_CH = 128


@functools.lru_cache(maxsize=None)
def _make(B, V, F, D):
    assert B % (_NW * _CH) == 0 and D % _L == 0
    b_per_w = B // _NW
    n_chunks = b_per_w // _CH
    mesh = plsc.VectorSubcoreMesh(core_axis_name="c", subcore_axis_name="s")

    @functools.partial(
        pl.kernel,
        out_type=jax.ShapeDtypeStruct((2 * B, D), jnp.float32),
        mesh=mesh,
        compiler_params=pltpu.CompilerParams(use_tc_tiling_on_sc=False),
        scratch_types=[
            pltpu.VMEM((n_chunks, _CH), jnp.int32),
            pltpu.VMEM((n_chunks, _CH), jnp.int32),
            pltpu.VMEM((n_chunks, _CH), jnp.int32),
            pltpu.VMEM((n_chunks, _CH), jnp.int32),
            pltpu.VMEM((n_chunks, _CH), jnp.int32),
            pltpu.VMEM((b_per_w, D), jnp.float32),
            pltpu.VMEM((b_per_w, D), jnp.float32),
            pltpu.SemaphoreType.DMA,
            pltpu.SemaphoreType.DMA,
        ],
    )
    def k(fp_idx_h, fb_idx_h, valid_h, fp_mat_h, fb_tab_h, out_h,
          fpi_v, fbi_v, val_v, pfp_v, pfb_v, rows_fp_v, rows_fb_v,
          sem_g, sem_s):
        wid = lax.axis_index("s") * _NC + lax.axis_index("c")
        cbase = wid * n_chunks
        pltpu.sync_copy(fp_idx_h.at[pl.ds(cbase, n_chunks)], fpi_v)
        pltpu.sync_copy(fb_idx_h.at[pl.ds(cbase, n_chunks)], fbi_v)
        pltpu.sync_copy(valid_h.at[pl.ds(cbase, n_chunks)], val_v)

        gathers = []
        for j in range(n_chunks):
            gathers.append(pltpu.async_copy(
                fp_mat_h.at[fpi_v.at[j]],
                rows_fp_v.at[pl.ds(j * _CH, _CH)], sem_g))
            gathers.append(pltpu.async_copy(
                fb_tab_h.at[fbi_v.at[j]],
                rows_fb_v.at[pl.ds(j * _CH, _CH)], sem_g))

        lane = lax.iota(jnp.int32, _L)
        for j in range(n_chunks):
            for c in range(_CH // _L):
                sl = (j, pl.ds(c * _L, _L))
                rows = (cbase + j) * _CH + c * _L + lane
                v = val_v[sl] != 0
                pfp_v[sl] = jnp.where(v, rows, B + rows)
                pfb_v[sl] = jnp.where(v, B + rows, rows)

        for g in gathers:
            g.wait()
        scatters = []
        for j in range(n_chunks):
            scatters.append(pltpu.async_copy(
                rows_fp_v.at[pl.ds(j * _CH, _CH)],
                out_h.at[pfp_v.at[j]], sem_s))
            scatters.append(pltpu.async_copy(
                rows_fb_v.at[pl.ds(j * _CH, _CH)],
                out_h.at[pfb_v.at[j]], sem_s))
        for s in scatters:
            s.wait()

    return k


@jax.jit
def kernel(fp_idx, fb_idx, is_valid, fingerprint_matrix, fallback_table):
    B = fp_idx.shape[0]
    V, D = fingerprint_matrix.shape
    F = fallback_table.shape[0]
    k = _make(B, V, F, D)
    fp2 = fp_idx.astype(jnp.int32).reshape(B // _CH, _CH)
    fb2 = fb_idx.astype(jnp.int32).reshape(B // _CH, _CH)
    va2 = is_valid.astype(jnp.int32).reshape(B // _CH, _CH)
    out_ext = k(fp2, fb2, va2, fingerprint_matrix, fallback_table)
    return out_ext[:B]
